# baseline (speedup 1.0000x reference)
import functools

import jax
import jax.numpy as jnp
from jax import lax
from jax.experimental import pallas as pl
from jax.experimental.pallas import tpu as pltpu
from jax.experimental.pallas import tpu_sc as plsc

_N_NODES = 10000
_D = 128
_E = 320000


def _normalize_body(z_ref, o_ref):
    x = z_ref[:]
    s = jnp.sum(x * x, axis=1, keepdims=True)
    o_ref[:] = (x * lax.rsqrt(s)).astype(jnp.bfloat16)


def _normalize(z):
    return pl.pallas_call(
        _normalize_body,
        out_shape=jax.ShapeDtypeStruct(z.shape, jnp.bfloat16),
    )(z)


def _make_edge_kernel():
    info = plsc.get_sparse_core_info()
    nc, ns, lanes = info.num_cores, info.num_subcores, info.num_lanes
    nw = nc * ns
    epw = _E // nw
    B = 80
    nch = epw // B
    nbuf = 2

    mesh = plsc.VectorSubcoreMesh(core_axis_name="c", subcore_axis_name="s")

    @functools.partial(
        pl.kernel,
        mesh=mesh,
        compiler_params=pltpu.CompilerParams(
            needs_layout_passes=False, use_tc_tiling_on_sc=False),
        out_type=jax.ShapeDtypeStruct((_E,), jnp.float32),
        scratch_types=[
            pltpu.VMEM((2 * epw,), jnp.int32),
            *[pltpu.VMEM((2 * B, _D // 2), jnp.int32)] * nbuf,
            pltpu.VMEM((epw,), jnp.float32),
            *[pltpu.SemaphoreType.DMA] * nbuf,
        ],
    )
    def edge_kernel(zn, idxi, out, iv, *rest):
        slots = rest[:nbuf]
        ov = rest[nbuf]
        sems = rest[nbuf + 1:]
        wid = lax.axis_index("s") * nc + lax.axis_index("c")
        base = pl.multiple_of(wid * epw, 8)

        pltpu.sync_copy(idxi.at[pl.ds(2 * base, 2 * epw)], iv)

        nstr = 4
        part = 2 * B // nstr

        def fire(c, rows, sem):
            start = pl.multiple_of(c * 2 * B, 8)
            for q in range(nstr):
                pltpu.async_copy(
                    zn.at[iv.at[pl.ds(start + q * part, part)]],
                    rows.at[pl.ds(q * part, part)], sem)

        def drain(c, rows, sem):
            start = pl.multiple_of(c * 2 * B, 8)
            for q in range(nstr):
                pltpu.make_async_copy(
                    zn.at[iv.at[pl.ds(start + q * part, part)]],
                    rows.at[pl.ds(q * part, part)], sem).wait()

        lane_ids = lax.iota(jnp.int32, lanes)

        bgroups = _D // (2 * lanes)

        def compute(c, rows):
            start = pl.multiple_of(c * B, 8)
            for g in range(B // lanes):
                outv = jnp.zeros((lanes,), jnp.float32)
                for l in range(lanes):
                    e = g * lanes + l
                    acc = None
                    for j in range(bgroups):
                        s = plsc.bitcast(
                            rows[2 * e, j * lanes:(j + 1) * lanes],
                            jnp.bfloat16)
                        d = plsc.bitcast(
                            rows[2 * e + 1, j * lanes:(j + 1) * lanes],
                            jnp.bfloat16)
                        p = s * d
                        acc = p if acc is None else acc + p
                    a, b = plsc.unpack(acc, format=plsc.PackFormat.INTERLEAVED)
                    tot = jnp.sum(a + b)
                    outv = jnp.where(lane_ids == l, tot, outv)
                ov[pl.ds(start + g * lanes, lanes)] = outv * 0.5 + 0.5

        for s in range(nbuf):
            fire(s, slots[s], sems[s])

        def loop_body(i, carry):
            c0 = i * nbuf
            for s in range(nbuf):
                c = c0 + s
                drain(c, slots[s], sems[s])
                compute(c, slots[s])

                @pl.when(c + nbuf < nch)
                def _(c=c, s=s):
                    fire(c + nbuf, slots[s], sems[s])

            return carry

        lax.fori_loop(0, nch // nbuf, loop_body, 0)

        for s in range(nch % nbuf):
            c = (nch // nbuf) * nbuf + s
            drain(c, slots[s], sems[s])
            compute(c, slots[s])

        pltpu.sync_copy(ov, out.at[pl.ds(base, epw)])

    return edge_kernel


_edge_kernel = _make_edge_kernel()


def kernel(z, edge_index):
    ei = edge_index.astype(jnp.int32)
    idx_il = jnp.swapaxes(ei, 0, 1).reshape(-1)
    zn = _normalize(z)
    zn_i32 = lax.bitcast_convert_type(
        zn.reshape(_N_NODES, _D // 2, 2), jnp.int32)
    return _edge_kernel(zn_i32, idx_il)


# speedup vs baseline: 1.7339x; 1.7339x over previous
import functools

import jax
import jax.numpy as jnp
from jax import lax
from jax.experimental import pallas as pl
from jax.experimental.pallas import tpu as pltpu
from jax.experimental.pallas import tpu_sc as plsc

_N_NODES = 10000
_D = 128
_E = 320000


def _normalize_body(z_ref, o_ref):
    x = z_ref[:]
    s = jnp.sum(x * x, axis=1, keepdims=True)
    o_ref[:] = (x * lax.rsqrt(s)).astype(jnp.bfloat16)


def _normalize(z):
    return pl.pallas_call(
        _normalize_body,
        out_shape=jax.ShapeDtypeStruct(z.shape, jnp.bfloat16),
    )(z)


def _make_edge_kernel():
    info = plsc.get_sparse_core_info()
    nc, ns, lanes = info.num_cores, info.num_subcores, info.num_lanes
    nw = nc * ns
    epw = _E // nw
    B = 80
    nch = epw // B
    nbuf = 2

    mesh = plsc.VectorSubcoreMesh(core_axis_name="c", subcore_axis_name="s")

    @functools.partial(
        pl.kernel,
        mesh=mesh,
        compiler_params=pltpu.CompilerParams(
            needs_layout_passes=False, use_tc_tiling_on_sc=False),
        out_type=jax.ShapeDtypeStruct((_E,), jnp.float32),
        scratch_types=[
            pltpu.VMEM((epw,), jnp.int32),
            pltpu.VMEM((epw,), jnp.int32),
            *[pltpu.VMEM((2 * B, _D // 2), jnp.int32)] * nbuf,
            pltpu.VMEM((epw,), jnp.float32),
            *[pltpu.SemaphoreType.DMA] * nbuf,
        ],
    )
    def edge_kernel(zn, srci, dsti, out, si, di, *rest):
        slots = rest[:nbuf]
        ov = rest[nbuf]
        sems = rest[nbuf + 1:]
        wid = lax.axis_index("s") * nc + lax.axis_index("c")
        base = pl.multiple_of(wid * epw, 8)

        pltpu.sync_copy(srci.at[pl.ds(base, epw)], si)
        pltpu.sync_copy(dsti.at[pl.ds(base, epw)], di)

        half = B // 2

        def fire(c, rows, sem):
            start = pl.multiple_of(c * B, 8)
            for q, iv in ((0, si), (1, di)):
                for h in range(2):
                    pltpu.async_copy(
                        zn.at[iv.at[pl.ds(start + h * half, half)]],
                        rows.at[pl.ds(q * B + h * half, half)], sem)

        def drain(c, rows, sem):
            start = pl.multiple_of(c * B, 8)
            for q, iv in ((0, si), (1, di)):
                for h in range(2):
                    pltpu.make_async_copy(
                        zn.at[iv.at[pl.ds(start + h * half, half)]],
                        rows.at[pl.ds(q * B + h * half, half)], sem).wait()

        lane_ids = lax.iota(jnp.int32, lanes)

        bgroups = _D // (2 * lanes)

        def compute(c, rows):
            start = pl.multiple_of(c * B, 8)
            for g in range(B // lanes):
                outv = jnp.zeros((lanes,), jnp.float32)
                for l in range(lanes):
                    e = g * lanes + l
                    acc = None
                    for j in range(bgroups):
                        s = plsc.bitcast(
                            rows[e, j * lanes:(j + 1) * lanes],
                            jnp.bfloat16)
                        d = plsc.bitcast(
                            rows[B + e, j * lanes:(j + 1) * lanes],
                            jnp.bfloat16)
                        p = s * d
                        acc = p if acc is None else acc + p
                    a, b = plsc.unpack(acc, format=plsc.PackFormat.INTERLEAVED)
                    tot = jnp.sum(a + b)
                    outv = jnp.where(lane_ids == l, tot, outv)
                ov[pl.ds(start + g * lanes, lanes)] = outv * 0.5 + 0.5

        for s in range(nbuf):
            fire(s, slots[s], sems[s])

        def loop_body(i, carry):
            c0 = i * nbuf
            for s in range(nbuf):
                c = c0 + s
                drain(c, slots[s], sems[s])
                compute(c, slots[s])

                @pl.when(c + nbuf < nch)
                def _(c=c, s=s):
                    fire(c + nbuf, slots[s], sems[s])

            return carry

        lax.fori_loop(0, nch // nbuf, loop_body, 0)

        for s in range(nch % nbuf):
            c = (nch // nbuf) * nbuf + s
            drain(c, slots[s], sems[s])
            compute(c, slots[s])

        pltpu.sync_copy(ov, out.at[pl.ds(base, epw)])

    return edge_kernel


_edge_kernel = _make_edge_kernel()


def kernel(z, edge_index):
    ei = edge_index.astype(jnp.int32)
    zn = _normalize(z)
    zn_i32 = lax.bitcast_convert_type(
        zn.reshape(_N_NODES, _D // 2, 2), jnp.int32)
    return _edge_kernel(zn_i32, ei[0], ei[1])
